# baseline (speedup 1.0000x reference)
import functools

import jax
import jax.numpy as jnp
from jax import lax
from jax.experimental import pallas as pl
from jax.experimental.pallas import tpu as pltpu
from jax.experimental.pallas import tpu_sc as plsc

N_CLASSES = 1000
B, F = 1024, 26
_INFO = plsc.get_sparse_core_info()
NC, NS = _INFO.num_cores, _INFO.num_subcores
NW = NC * NS
B_PER_W = B // NW


@functools.partial(
    pl.kernel,
    mesh=plsc.VectorSubcoreMesh(core_axis_name="c", subcore_axis_name="s"),
    out_type=jax.ShapeDtypeStruct((B, F, N_CLASSES), jnp.int32),
    scratch_types=[
        pltpu.VMEM((B_PER_W * F,), jnp.int32),
        pltpu.VMEM((1, F, N_CLASSES), jnp.int32),
        pltpu.VMEM((1, F, N_CLASSES), jnp.int32),
        pltpu.SemaphoreType.DMA,
        pltpu.SemaphoreType.DMA,
    ],
    compiler_params=pltpu.CompilerParams(needs_layout_passes=False),
)
def _one_hot_sc(x_hbm, out_hbm, idx_v, buf_a, buf_b, sem_a, sem_b):
    wid = lax.axis_index("s") * NC + lax.axis_index("c")
    base_b = wid * B_PER_W
    pltpu.sync_copy(x_hbm.at[pl.ds(base_b * F, B_PER_W * F)], idx_v)

    zeros16 = jnp.zeros((16,), jnp.int32)
    ones16 = jnp.ones((16,), jnp.int32)
    lane = lax.iota(jnp.int32, 16)
    bufs = (buf_a, buf_b)
    sems = (sem_a, sem_b)

    for buf in bufs:
        def zero_body(ff, carry, buf=buf):
            for k in range(N_CLASSES // 16):
                buf[0, ff, pl.ds(k * 16, 16)] = zeros16
            buf[0, ff, pl.ds(N_CLASSES - 16, 16)] = zeros16
            return carry

        lax.fori_loop(0, F, zero_body, 0)

    def scatter_batch(buf, c, value16):
        for v in range(2):
            r0 = v * 16
            ff = lane + r0
            mask = ff < F
            idx = idx_v[pl.ds(c * F + r0, 16)]
            plsc.store_scatter(buf, [jnp.zeros((16,), jnp.int32), ff, idx],
                               value16, mask=mask)

    def start_dma(buf, sem, c):
        pltpu.async_copy(buf, out_hbm.at[pl.ds(base_b + c, 1)], sem)

    def wait_dma(buf, sem, c):
        pltpu.make_async_copy(buf, out_hbm.at[pl.ds(base_b + c, 1)], sem).wait()

    for j in range(2):
        scatter_batch(bufs[j], j, ones16)
        start_dma(bufs[j], sems[j], j)

    def pair_body(c0, carry):
        for j in range(2):
            c = c0 + j
            buf, sem = bufs[j], sems[j]
            wait_dma(buf, sem, c - 2)
            scatter_batch(buf, c - 2, zeros16)
            scatter_batch(buf, c, ones16)
            start_dma(buf, sem, c)
        return carry

    lax.fori_loop(1, B_PER_W // 2, lambda i, cr: pair_body(i * 2, cr), 0)

    for j in range(2):
        wait_dma(bufs[j], sems[j], B_PER_W - 2 + j)


def kernel(x):
    return _one_hot_sc(x.reshape(B * F))


# speedup vs baseline: 1.1193x; 1.1193x over previous
import jax
import jax.numpy as jnp
from jax import lax
from jax.experimental import pallas as pl

N_CLASSES = 1000
B, F = 1024, 26
_BT = 64


def _tc_body(x_ref, out_ref):
    x = x_ref[...]
    iota = lax.broadcasted_iota(jnp.int32, (_BT, F, N_CLASSES), 2)
    out_ref[...] = (x[..., None] == iota).astype(jnp.int32)


_tc_one_hot = pl.pallas_call(
    _tc_body,
    out_shape=jax.ShapeDtypeStruct((B, F, N_CLASSES), jnp.int32),
    grid=(B // _BT,),
    in_specs=[pl.BlockSpec((_BT, F), lambda i: (i, 0))],
    out_specs=pl.BlockSpec((_BT, F, N_CLASSES), lambda i: (i, 0, 0)),
)


def kernel(x):
    return _tc_one_hot(x)
